# baseline (speedup 1.0000x reference)
import functools

import jax
import jax.numpy as jnp
from jax import lax
from jax.experimental import pallas as pl
from jax.experimental.pallas import tpu as pltpu
from jax.experimental.pallas import tpu_sc as plsc

N = 10000
D = 128
E = 320000

NUM_CORES = 2
NUM_SUBCORES = 16
NW = NUM_CORES * NUM_SUBCORES
EPW = E // NW
LANES = 16

BM = 400


def _tc_body(adj_ref, x_ref, wenc_ref, w2_ref, bias_ref, out_ref):
    h = jnp.dot(adj_ref[...], x_ref[...], preferred_element_type=jnp.float32)
    h = h * (1.0 / N)
    e = jnp.maximum(jnp.dot(h, wenc_ref[...], preferred_element_type=jnp.float32), 0.0)
    out_ref[...] = jnp.dot(e, w2_ref[...], preferred_element_type=jnp.float32) + bias_ref[...]


def _node_scalars(adj, x, W_enc, W2, bvec):
    return pl.pallas_call(
        _tc_body,
        grid=(N // BM,),
        in_specs=[
            pl.BlockSpec((BM, N), lambda i: (i, 0)),
            pl.BlockSpec((N, D), lambda i: (0, 0)),
            pl.BlockSpec((D, D), lambda i: (0, 0)),
            pl.BlockSpec((D, 2), lambda i: (0, 0)),
            pl.BlockSpec((1, 2), lambda i: (0, 0)),
        ],
        out_specs=pl.BlockSpec((BM, 2), lambda i: (i, 0)),
        out_shape=jax.ShapeDtypeStruct((N, 2), jnp.float32),
        compiler_params=pltpu.CompilerParams(
            dimension_semantics=("arbitrary",),
            vmem_limit_bytes=110 * 1024 * 1024,
        ),
    )(adj, x, W_enc, W2, bvec)


def _edge_scores_body(st_hbm, src_hbm, dst_hbm, out_hbm, st_v, src_v, dst_v, o_v, sem):
    wid = lax.axis_index("s") * NUM_CORES + lax.axis_index("c")
    base = wid * EPW
    c_st = pltpu.async_copy(st_hbm, st_v, sem)
    c_src = pltpu.async_copy(src_hbm.at[pl.ds(base, EPW)], src_v, sem)
    c_dst = pltpu.async_copy(dst_hbm.at[pl.ds(base, EPW)], dst_v, sem)
    c_st.wait()
    c_src.wait()
    c_dst.wait()

    @plsc.parallel_loop(0, EPW, step=LANES, unroll=8)
    def body(off):
        si = src_v[pl.ds(off, LANES)] * 2
        di = dst_v[pl.ds(off, LANES)] * 2 + 1
        vs = plsc.load_gather(st_v, [si])
        vt = plsc.load_gather(st_v, [di])
        o_v[pl.ds(off, LANES)] = vs + vt

    pltpu.sync_copy(o_v, out_hbm.at[pl.ds(base, EPW)])


@functools.lru_cache(maxsize=1)
def _edge_scores_kernel():
    mesh = plsc.VectorSubcoreMesh(
        core_axis_name="c", subcore_axis_name="s",
        num_cores=NUM_CORES, num_subcores=NUM_SUBCORES,
    )
    return pl.kernel(
        _edge_scores_body,
        out_type=jax.ShapeDtypeStruct((E,), jnp.float32),
        mesh=mesh,
        scratch_types=[
            pltpu.VMEM((N * 2,), jnp.float32),
            pltpu.VMEM((EPW,), jnp.int32),
            pltpu.VMEM((EPW,), jnp.int32),
            pltpu.VMEM((EPW,), jnp.float32),
            pltpu.SemaphoreType.DMA,
        ],
        compiler_params=pltpu.CompilerParams(needs_layout_passes=False),
    )


def kernel(x, adj, edge_index, W_enc, W_mlp, b_mlp):
    W2 = jnp.concatenate([W_mlp[:D], W_mlp[D:]], axis=1)
    bvec = jnp.concatenate([b_mlp, jnp.zeros((1,), jnp.float32)]).reshape(1, 2)
    st = _node_scalars(adj, x, W_enc, W2, bvec)
    out = _edge_scores_kernel()(st.reshape(-1), edge_index[0], edge_index[1])
    return out.reshape(E, 1)


# speedup vs baseline: 1.0541x; 1.0541x over previous
import functools

import jax
import jax.numpy as jnp
from jax import lax
from jax.experimental import pallas as pl
from jax.experimental.pallas import tpu as pltpu
from jax.experimental.pallas import tpu_sc as plsc

N = 10000
D = 128
E = 320000

NUM_CORES = 2
NUM_SUBCORES = 16
NW = NUM_CORES * NUM_SUBCORES
EPW = E // NW
LANES = 16

BM = 400


def _tc_body(adj_ref, x_ref, wenc_ref, w2_ref, bias_ref, out_ref):
    h = jnp.dot(adj_ref[...], x_ref[...], preferred_element_type=jnp.float32)
    h = h * (1.0 / N)
    e = jnp.maximum(jnp.dot(h, wenc_ref[...], preferred_element_type=jnp.float32), 0.0)
    out_ref[...] = jnp.dot(e, w2_ref[...], preferred_element_type=jnp.float32) + bias_ref[...]


def _node_scalars(adj, x, W_enc, W2, bvec):
    return pl.pallas_call(
        _tc_body,
        grid=(N // BM,),
        in_specs=[
            pl.BlockSpec((BM, N), lambda i: (i, 0)),
            pl.BlockSpec((N, D), lambda i: (0, 0)),
            pl.BlockSpec((D, D), lambda i: (0, 0)),
            pl.BlockSpec((D, 2), lambda i: (0, 0)),
            pl.BlockSpec((1, 2), lambda i: (0, 0)),
        ],
        out_specs=pl.BlockSpec((BM, 2), lambda i: (i, 0)),
        out_shape=jax.ShapeDtypeStruct((N, 2), jnp.float32),
        compiler_params=pltpu.CompilerParams(
            dimension_semantics=("arbitrary",),
            vmem_limit_bytes=110 * 1024 * 1024,
        ),
    )(adj, x, W_enc, W2, bvec)


def _edge_scores_body(st_hbm, edges_hbm, out_hbm, st_v, src_v, dst_v, o_v, sem):
    wid = lax.axis_index("s") * NUM_CORES + lax.axis_index("c")
    base = wid * EPW
    c_st = pltpu.async_copy(st_hbm, st_v, sem)
    c_src = pltpu.async_copy(edges_hbm.at[pl.ds(base, EPW)], src_v, sem)
    c_dst = pltpu.async_copy(edges_hbm.at[pl.ds(E + base, EPW)], dst_v, sem)
    c_st.wait()
    c_src.wait()
    c_dst.wait()

    @plsc.parallel_loop(0, EPW, step=LANES, unroll=8)
    def body(off):
        si = src_v[pl.ds(off, LANES)] * 2
        di = dst_v[pl.ds(off, LANES)] * 2 + 1
        vs = plsc.load_gather(st_v, [si])
        vt = plsc.load_gather(st_v, [di])
        o_v[pl.ds(off, LANES)] = vs + vt

    pltpu.sync_copy(o_v, out_hbm.at[pl.ds(base, EPW)])


@functools.lru_cache(maxsize=1)
def _edge_scores_kernel():
    mesh = plsc.VectorSubcoreMesh(
        core_axis_name="c", subcore_axis_name="s",
        num_cores=NUM_CORES, num_subcores=NUM_SUBCORES,
    )
    return pl.kernel(
        _edge_scores_body,
        out_type=jax.ShapeDtypeStruct((E,), jnp.float32),
        mesh=mesh,
        scratch_types=[
            pltpu.VMEM((N * 2,), jnp.float32),
            pltpu.VMEM((EPW,), jnp.int32),
            pltpu.VMEM((EPW,), jnp.int32),
            pltpu.VMEM((EPW,), jnp.float32),
            pltpu.SemaphoreType.DMA,
        ],
        compiler_params=pltpu.CompilerParams(needs_layout_passes=False),
    )


def kernel(x, adj, edge_index, W_enc, W_mlp, b_mlp):
    W2 = jnp.concatenate([W_mlp[:D], W_mlp[D:]], axis=1)
    bvec = jnp.concatenate([b_mlp, jnp.zeros((1,), jnp.float32)]).reshape(1, 2)
    st = _node_scalars(adj, x, W_enc, W2, bvec)
    out = _edge_scores_kernel()(st.reshape(-1), edge_index.reshape(-1))
    return out.reshape(E, 1)


# speedup vs baseline: 1.0609x; 1.0065x over previous
import functools

import jax
import jax.numpy as jnp
from jax import lax
from jax.experimental import pallas as pl
from jax.experimental.pallas import tpu as pltpu
from jax.experimental.pallas import tpu_sc as plsc

N = 10000
D = 128
E = 320000

NUM_CORES = 2
NUM_SUBCORES = 16
NW = NUM_CORES * NUM_SUBCORES
EPW = E // NW
LANES = 16

BM = 400


def _tc_body(adj_ref, x_ref, wenc_ref, wmlp_ref, b_ref, out_ref):
    h = jnp.dot(adj_ref[...], x_ref[...], preferred_element_type=jnp.float32)
    h = h * (1.0 / N)
    e = jnp.maximum(jnp.dot(h, wenc_ref[...], preferred_element_type=jnp.float32), 0.0)
    w2 = jnp.concatenate([wmlp_ref[0:D, :], wmlp_ref[D:, :]], axis=1)
    bias = jnp.concatenate([b_ref[...], jnp.zeros((1, 1), jnp.float32)], axis=1)
    out_ref[...] = jnp.dot(e, w2, preferred_element_type=jnp.float32) + bias


def _node_scalars(adj, x, W_enc, W_mlp, b_mlp):
    return pl.pallas_call(
        _tc_body,
        grid=(N // BM,),
        in_specs=[
            pl.BlockSpec((BM, N), lambda i: (i, 0)),
            pl.BlockSpec((N, D), lambda i: (0, 0)),
            pl.BlockSpec((D, D), lambda i: (0, 0)),
            pl.BlockSpec((2 * D, 1), lambda i: (0, 0)),
            pl.BlockSpec((1, 1), lambda i: (0, 0)),
        ],
        out_specs=pl.BlockSpec((BM, 2), lambda i: (i, 0)),
        out_shape=jax.ShapeDtypeStruct((N, 2), jnp.float32),
        compiler_params=pltpu.CompilerParams(
            dimension_semantics=("arbitrary",),
            vmem_limit_bytes=110 * 1024 * 1024,
        ),
    )(adj, x, W_enc, W_mlp, b_mlp.reshape(1, 1))


def _edge_scores_body(st_hbm, edges_hbm, out_hbm, st_v, src_v, dst_v, o_v, sem):
    wid = lax.axis_index("s") * NUM_CORES + lax.axis_index("c")
    base = wid * EPW
    c_st = pltpu.async_copy(st_hbm, st_v, sem)
    c_src = pltpu.async_copy(edges_hbm.at[pl.ds(base, EPW)], src_v, sem)
    c_dst = pltpu.async_copy(edges_hbm.at[pl.ds(E + base, EPW)], dst_v, sem)
    c_st.wait()
    c_src.wait()
    c_dst.wait()

    @plsc.parallel_loop(0, EPW, step=LANES, unroll=8)
    def body(off):
        si = src_v[pl.ds(off, LANES)] * 2
        di = dst_v[pl.ds(off, LANES)] * 2 + 1
        vs = plsc.load_gather(st_v, [si])
        vt = plsc.load_gather(st_v, [di])
        o_v[pl.ds(off, LANES)] = vs + vt

    pltpu.sync_copy(o_v, out_hbm.at[pl.ds(base, EPW)])


@functools.lru_cache(maxsize=1)
def _edge_scores_kernel():
    mesh = plsc.VectorSubcoreMesh(
        core_axis_name="c", subcore_axis_name="s",
        num_cores=NUM_CORES, num_subcores=NUM_SUBCORES,
    )
    return pl.kernel(
        _edge_scores_body,
        out_type=jax.ShapeDtypeStruct((E,), jnp.float32),
        mesh=mesh,
        scratch_types=[
            pltpu.VMEM((N * 2,), jnp.float32),
            pltpu.VMEM((EPW,), jnp.int32),
            pltpu.VMEM((EPW,), jnp.int32),
            pltpu.VMEM((EPW,), jnp.float32),
            pltpu.SemaphoreType.DMA,
        ],
        compiler_params=pltpu.CompilerParams(needs_layout_passes=False),
    )


def kernel(x, adj, edge_index, W_enc, W_mlp, b_mlp):
    st = _node_scalars(adj, x, W_enc, W_mlp, b_mlp)
    out = _edge_scores_kernel()(st.reshape(-1), edge_index.reshape(-1))
    return out.reshape(E, 1)
